# Initial kernel scaffold; baseline (speedup 1.0000x reference)
import jax
import jax.numpy as jnp
from jax.experimental import pallas as pl


def kernel(x, table):
    raise NotImplementedError("write your pallas kernel here")


# speedup vs baseline: 3.9488x; 3.9488x over previous
import functools

import jax
import jax.numpy as jnp
from jax import lax
from jax.experimental import pallas as pl
from jax.experimental.pallas import tpu as pltpu
from jax.experimental.pallas import tpu_sc as plsc

D_MODEL = 64
NC = 2
NS = 16
NW = NC * NS
CH = 512


def _embed(xf, table):
    B = xf.shape[0]
    b_per_w = B // NW
    n_ch = b_per_w // CH
    mesh = plsc.VectorSubcoreMesh(
        core_axis_name="c", subcore_axis_name="s", num_cores=NC, num_subcores=NS
    )

    @functools.partial(
        pl.kernel,
        out_type=jax.ShapeDtypeStruct((B, D_MODEL), jnp.float32),
        mesh=mesh,
        scratch_types=[
            pltpu.VMEM((CH,), jnp.int32),
            pltpu.VMEM((CH, D_MODEL), jnp.float32),
            pltpu.SemaphoreType.DMA,
        ],
        compiler_params=pltpu.CompilerParams(use_tc_tiling_on_sc=False),
    )
    def k(x_hbm, tab_hbm, out_hbm, idx_v, rows_v, sem):
        wid = lax.axis_index("s") * NC + lax.axis_index("c")
        base = wid * b_per_w

        def body(i, carry):
            off = base + i * CH
            pltpu.sync_copy(x_hbm.at[pl.ds(off, CH)], idx_v)
            pltpu.async_copy(tab_hbm.at[idx_v], rows_v, sem).wait()
            pltpu.sync_copy(rows_v, out_hbm.at[pl.ds(off, CH)])
            return carry

        lax.fori_loop(0, n_ch, body, 0)

    return k(xf, table)


def kernel(x, table):
    b, s = x.shape
    xf = x.reshape(b * s).astype(jnp.int32)
    out = _embed(xf, table)
    return out.reshape(b, s, D_MODEL)


# speedup vs baseline: 4.2686x; 1.0810x over previous
import functools

import jax
import jax.numpy as jnp
from jax import lax
from jax.experimental import pallas as pl
from jax.experimental.pallas import tpu as pltpu
from jax.experimental.pallas import tpu_sc as plsc

D_MODEL = 64
NC = 2
NS = 16
NW = NC * NS
CH = 512


def _embed(xf, table):
    B = xf.shape[0]
    b_per_w = B // NW
    n_ch = b_per_w // CH
    assert b_per_w % CH == 0 and n_ch >= 4 and n_ch % 2 == 0
    mesh = plsc.VectorSubcoreMesh(
        core_axis_name="c", subcore_axis_name="s", num_cores=NC, num_subcores=NS
    )

    @functools.partial(
        pl.kernel,
        out_type=jax.ShapeDtypeStruct((B, D_MODEL), jnp.float32),
        mesh=mesh,
        scratch_types=[
            pltpu.VMEM((CH,), jnp.int32),
            pltpu.VMEM((CH,), jnp.int32),
            pltpu.VMEM((CH, D_MODEL), jnp.float32),
            pltpu.VMEM((CH, D_MODEL), jnp.float32),
            pltpu.SemaphoreType.DMA,
            pltpu.SemaphoreType.DMA,
            pltpu.SemaphoreType.DMA,
            pltpu.SemaphoreType.DMA,
            pltpu.SemaphoreType.DMA,
            pltpu.SemaphoreType.DMA,
        ],
        compiler_params=pltpu.CompilerParams(use_tc_tiling_on_sc=False),
    )
    def k(x_hbm, tab_hbm, out_hbm, idx0, idx1, rows0, rows1,
          is0, is1, gs0, gs1, ss0, ss1):
        idx_v = (idx0, idx1)
        rows_v = (rows0, rows1)
        isem = (is0, is1)
        gsem = (gs0, gs1)
        ssem = (ss0, ss1)
        wid = lax.axis_index("s") * NC + lax.axis_index("c")
        base = wid * b_per_w

        def start_idx(j, b):
            pltpu.async_copy(
                x_hbm.at[pl.ds(base + j * CH, CH)], idx_v[b], isem[b])

        def wait_idx(b):
            pltpu.make_async_copy(
                x_hbm.at[pl.ds(base, CH)], idx_v[b], isem[b]).wait()

        def start_gather(b):
            pltpu.async_copy(tab_hbm.at[idx_v[b]], rows_v[b], gsem[b])

        def wait_gather(b):
            pltpu.make_async_copy(
                tab_hbm.at[idx_v[b]], rows_v[b], gsem[b]).wait()

        def start_store(j, b):
            pltpu.async_copy(
                rows_v[b], out_hbm.at[pl.ds(base + j * CH, CH)], ssem[b])

        def wait_store(b):
            pltpu.make_async_copy(
                rows_v[b], out_hbm.at[pl.ds(base, CH)], ssem[b]).wait()

        start_idx(0, 0)
        wait_idx(0)
        start_gather(0)
        start_idx(1, 1)
        wait_idx(1)
        start_gather(1)
        wait_gather(0)
        start_store(0, 0)
        start_idx(2, 0)

        @pl.loop(2, n_ch - 2, step=2)
        def _(t):
            for b in (0, 1):
                j = t + b
                pb = 1 - b
                wait_store(b)
                wait_idx(b)
                start_gather(b)
                wait_gather(pb)
                start_store(j - 1, pb)
                start_idx(j + 1, pb)

        for j in (n_ch - 2, n_ch - 1):
            b = j % 2
            pb = 1 - b
            wait_store(b)
            wait_idx(b)
            start_gather(b)
            wait_gather(pb)
            start_store(j - 1, pb)
            if j + 1 < n_ch:
                start_idx(j + 1, pb)
        last = (n_ch - 1) % 2
        wait_gather(last)
        start_store(n_ch - 1, last)
        wait_store(1 - last)
        wait_store(last)

    return k(xf, table)


def kernel(x, table):
    b, s = x.shape
    xf = x.reshape(b * s).astype(jnp.int32)
    out = _embed(xf, table)
    return out.reshape(b, s, D_MODEL)
